# Initial kernel scaffold; baseline (speedup 1.0000x reference)
import jax
import jax.numpy as jnp
from jax.experimental import pallas as pl


def kernel(x, pos_embedding, token_embedding):
    raise NotImplementedError("write your pallas kernel here")


# speedup vs baseline: 2.2343x; 2.2343x over previous
import functools

import jax
import jax.numpy as jnp
from jax import lax
from jax.experimental import pallas as pl
from jax.experimental.pallas import tpu as pltpu
from jax.experimental.pallas import tpu_sc as plsc

NC = 2
NS = 16
L = 16
NW = NC * NS


def kernel(x, pos_embedding, token_embedding):
    B, S = x.shape
    V, D = token_embedding.shape
    N = B * S
    per_w = N // NW
    CH = 128
    n_ch = per_w // CH

    x3 = x.reshape(NW, n_ch, CH).astype(jnp.int32)

    mesh = plsc.VectorSubcoreMesh(core_axis_name="c", subcore_axis_name="s")

    @functools.partial(
        pl.kernel,
        out_type=jax.ShapeDtypeStruct((N, D), jnp.float32),
        mesh=mesh,
        scratch_types=[
            pltpu.VMEM((n_ch, CH), jnp.int32),
            pltpu.VMEM((CH, D), jnp.float32),
            pltpu.VMEM((CH, D), jnp.float32),
            pltpu.SemaphoreType.DMA,
            pltpu.SemaphoreType.DMA,
        ],
    )
    def run(x_hbm, pos_hbm, tok_hbm, out_hbm, idx_v, tok_v, pos_v, gsem, psem):
        wid = lax.axis_index("s") * NC + lax.axis_index("c")
        base = wid * per_w
        pos_base = lax.rem(base, S)

        pltpu.sync_copy(x_hbm.at[wid], idx_v)

        for c in range(n_ch):
            gcp = pltpu.async_copy(tok_hbm.at[idx_v.at[c]], tok_v, gsem)
            pcp = pltpu.async_copy(
                pos_hbm.at[pl.ds(pos_base + c * CH, CH)], pos_v, psem)
            gcp.wait()
            pcp.wait()

            def row_add(r, carry):
                for k in range(D // L):
                    sl = pl.ds(k * L, L)
                    tok_v[r, sl] = tok_v[r, sl] + pos_v[r, sl]
                return carry

            lax.fori_loop(0, CH, row_add, 0)

            pltpu.sync_copy(tok_v, out_hbm.at[pl.ds(base + c * CH, CH)])

    out = run(x3, pos_embedding, token_embedding)
    return out.reshape(B, S, D)


# speedup vs baseline: 2.7933x; 1.2502x over previous
import functools

import jax
import jax.numpy as jnp
from jax import lax
from jax.experimental import pallas as pl
from jax.experimental.pallas import tpu as pltpu
from jax.experimental.pallas import tpu_sc as plsc

NC = 2
NS = 16
L = 16
NW = NC * NS


def kernel(x, pos_embedding, token_embedding):
    B, S = x.shape
    V, D = token_embedding.shape
    CH = S // NW

    xt = x.astype(jnp.int32).reshape(B, NW, CH).transpose(1, 0, 2)

    mesh = plsc.VectorSubcoreMesh(core_axis_name="c", subcore_axis_name="s")

    @functools.partial(
        pl.kernel,
        out_type=jax.ShapeDtypeStruct((B * S, D), jnp.float32),
        mesh=mesh,
        scratch_types=[
            pltpu.VMEM((B, CH), jnp.int32),
            pltpu.VMEM((CH, D), jnp.float32),
            pltpu.VMEM((CH, D), jnp.float32),
            pltpu.VMEM((CH, D), jnp.float32),
            pltpu.SemaphoreType.DMA,
            pltpu.SemaphoreType.DMA,
            pltpu.SemaphoreType.DMA,
            pltpu.SemaphoreType.DMA,
            pltpu.SemaphoreType.DMA,
        ],
    )
    def run(x_hbm, pos_hbm, tok_hbm, out_hbm,
            idx_v, pos_v, tok0, tok1, g0, g1, st0, st1, psem):
        wid = lax.axis_index("s") * NC + lax.axis_index("c")
        pbase = wid * CH

        pltpu.sync_copy(x_hbm.at[wid], idx_v)
        toks = [tok0, tok1]
        gsems = [g0, g1]
        ssems = [st0, st1]

        gcp = {0: pltpu.async_copy(tok_hbm.at[idx_v.at[0]], toks[0], gsems[0])}
        pltpu.async_copy(pos_hbm.at[pl.ds(pbase, CH)], pos_v, psem).wait()

        stcp = {}
        for b in range(B):
            cur = b % 2
            if b + 1 < B:
                if b >= 1:
                    stcp[b - 1].wait()
                gcp[b + 1] = pltpu.async_copy(
                    tok_hbm.at[idx_v.at[b + 1]], toks[1 - cur], gsems[1 - cur])
            gcp[b].wait()

            tok_v = toks[cur]

            def row_add(r, carry):
                for k in range(D // L):
                    sl = pl.ds(k * L, L)
                    tok_v[r, sl] = tok_v[r, sl] + pos_v[r, sl]
                return carry

            lax.fori_loop(0, CH, row_add, 0)

            stcp[b] = pltpu.async_copy(
                tok_v, out_hbm.at[pl.ds(b * S + pbase, CH)], ssems[cur])

        stcp[B - 2].wait()
        stcp[B - 1].wait()

    out = run(xt, pos_embedding, token_embedding)
    return out.reshape(B, S, D)
